# Initial kernel scaffold; baseline (speedup 1.0000x reference)
import jax
import jax.numpy as jnp
from jax.experimental import pallas as pl


def kernel(index_scores):
    raise NotImplementedError("write your pallas kernel here")


# speedup vs baseline: 7.0961x; 7.0961x over previous
import functools

import numpy as np
import jax
import jax.numpy as jnp
from jax.experimental import pallas as pl

B, Q, K = 4, 4096, 4096
KSEL = 512
R = 32

_nb = int(np.float32(-1e9).view(np.int32))
KM = _nb ^ 0x7FFFFFFF if _nb < 0 else _nb
INT32_MIN = -(2 ** 31)


def _rollL(x, s):
    return jnp.concatenate([x[:, s:], x[:, :s]], axis=1)


def _rollR(x, s):
    return jnp.concatenate([x[:, -s:], x[:, :-s]], axis=1)


def _topk_block_kernel(scores_ref, mask_ref, idx_ref, *, P, q0):
    i = pl.program_id(1)
    q = q0 + i * R + jax.lax.broadcasted_iota(jnp.int32, (R, 1), 0)
    cols = jax.lax.broadcasted_iota(jnp.int32, (R, P), 1)

    sc = scores_ref[0]
    b = jax.lax.bitcast_convert_type(sc, jnp.int32)
    key = jnp.where(b < 0, b ^ jnp.int32(0x7FFFFFFF), b)
    key = jnp.where(cols > q, jnp.int32(KM), key)

    cnt0 = jnp.sum((key >= 0).astype(jnp.int32), axis=1, keepdims=True)
    prefix = jnp.where(cnt0 >= KSEL, jnp.int32(0), jnp.int32(INT32_MIN))

    def _bsearch_step(it, prefix):
        t = prefix + (jnp.int32(1) << (jnp.int32(30) - it))
        cnt = jnp.sum((key >= t).astype(jnp.int32), axis=1, keepdims=True)
        return jnp.where(cnt >= KSEL, t, prefix)

    tau = jax.lax.fori_loop(0, 31, _bsearch_step, prefix)

    gt = key > tau
    eq = key == tau
    m = jnp.sum(gt.astype(jnp.int32), axis=1, keepdims=True)

    z = gt.astype(jnp.int32) * 65536 + eq.astype(jnp.int32)
    s = 1
    while s < P:
        z = z + jnp.where(cols >= s, _rollR(z, s), 0)
        s *= 2
    pg = z >> 16
    pe = z & 0xFFFF
    need = KSEL - m
    sel = gt | (eq & (pe <= need))
    mask_ref[0] = sel.astype(jnp.uint8)

    pos0 = pg + jnp.minimum(pe, need) - 1
    dist = jnp.where(sel, cols - pos0, 0)
    d2 = (dist << 1) | sel.astype(jnp.int32)
    idx = cols
    s = 1
    while s < P:
        rd2 = _rollL(d2, s)
        take = ((rd2 & 1) == 1) & (((rd2 >> 1) & s) != 0)
        d2 = jnp.where(take, rd2 & ~(2 * s), d2)
        key = jnp.where(take, _rollL(key, s), key)
        idx = jnp.where(take, _rollL(idx, s), idx)
        moved = ((d2 & 1) == 1) & (((d2 >> 1) & s) != 0) & (~take)
        d2 = jnp.where(moved, 0, d2)
        s *= 2
    skey = key[:, :KSEL]
    sidx = idx[:, :KSEL]

    colk = jax.lax.broadcasted_iota(jnp.int32, (R, KSEL), 1)
    for p in range(1, 10):
        for j in range(p - 1, -1, -1):
            sft = 1 << j
            lower = (colk & sft) == 0
            pk = jnp.where(lower, _rollL(skey, sft), _rollR(skey, sft))
            pi = jnp.where(lower, _rollL(sidx, sft), _rollR(sidx, sft))
            g = (pk > skey) | ((pk == skey) & (pi < sidx))
            take = g ^ (~lower)
            if p < 9:
                take = take ^ (((colk >> p) & 1) == 1)
            skey = jnp.where(take, pk, skey)
            sidx = jnp.where(take, pi, sidx)
    idx_ref[0] = sidx


def _run_slab(index_scores, q0, rows, P):
    grid = (B, rows // R)
    kern = functools.partial(_topk_block_kernel, P=P, q0=q0)
    mask_u8, idx = pl.pallas_call(
        kern,
        grid=grid,
        in_specs=[
            pl.BlockSpec((1, R, P), lambda b, i, q0r=q0 // R: (b, q0r + i, 0)),
        ],
        out_specs=[
            pl.BlockSpec((1, R, P), lambda b, i: (b, i, 0)),
            pl.BlockSpec((1, R, KSEL), lambda b, i: (b, i, 0)),
        ],
        out_shape=[
            jax.ShapeDtypeStruct((B, rows, P), jnp.uint8),
            jax.ShapeDtypeStruct((B, rows, KSEL), jnp.int32),
        ],
    )(index_scores)
    if P < K:
        mask_u8 = jnp.concatenate(
            [mask_u8, jnp.zeros((B, rows, K - P), jnp.uint8)], axis=2)
    return mask_u8, idx


def kernel(index_scores):
    slabs = [(0, 512, 512), (512, 512, 1024), (1024, 1024, 2048),
             (2048, 2048, 4096)]
    mask_parts, idx_parts = [], []
    for q0, rows, P in slabs:
        m_u8, idx = _run_slab(index_scores, q0, rows, P)
        mask_parts.append(m_u8)
        idx_parts.append(idx)
    top_k_mask = jnp.concatenate(mask_parts, axis=1) != 0
    top_k_indices = jnp.concatenate(idx_parts, axis=1)
    stats = jnp.asarray(
        [1.0 - KSEL / K, float(KSEL), KSEL / K], dtype=jnp.float32)
    return (top_k_mask, top_k_indices, stats)


# speedup vs baseline: 8.8502x; 1.2472x over previous
import functools

import numpy as np
import jax
import jax.numpy as jnp
from jax.experimental import pallas as pl

B, Q, K = 4, 4096, 4096
KSEL = 512
R = 128

_nb = int(np.float32(-1e9).view(np.int32))
KM = _nb ^ 0x7FFFFFFF if _nb < 0 else _nb
INT32_MIN = -(2 ** 31)


def _rollL(x, s):
    return jnp.concatenate([x[:, s:], x[:, :s]], axis=1)


def _rollR(x, s):
    return jnp.concatenate([x[:, -s:], x[:, :-s]], axis=1)


def _topk_block_kernel(scores_ref, mask_ref, idx_ref, *, P, q0):
    i = pl.program_id(1)
    q = q0 + i * R + jax.lax.broadcasted_iota(jnp.int32, (R, 1), 0)
    cols = jax.lax.broadcasted_iota(jnp.int32, (R, P), 1)

    sc = scores_ref[0]
    b = jax.lax.bitcast_convert_type(sc, jnp.int32)
    key = jnp.where(b < 0, b ^ jnp.int32(0x7FFFFFFF), b)
    key = jnp.where(cols > q, jnp.int32(KM) - cols, key)

    cnt0 = jnp.sum((key >= 0).astype(jnp.int32), axis=1, keepdims=True)
    prefix = jnp.where(cnt0 >= KSEL, jnp.int32(0), jnp.int32(INT32_MIN))

    def _bsearch_step(it, prefix):
        t = prefix + (jnp.int32(1) << (jnp.int32(30) - it))
        cnt = jnp.sum((key >= t).astype(jnp.int32), axis=1, keepdims=True)
        return jnp.where(cnt >= KSEL, t, prefix)

    tau = jax.lax.fori_loop(0, 31, _bsearch_step, prefix)

    gt = key > tau
    eq = key == tau
    m = jnp.sum(gt.astype(jnp.int32), axis=1, keepdims=True)

    z = gt.astype(jnp.int32) * 65536 + eq.astype(jnp.int32)
    s = 1
    while s < P:
        z = z + jnp.where(cols >= s, _rollR(z, s), 0)
        s *= 2
    pg = z >> 16
    pe = z & 0xFFFF
    need = KSEL - m
    sel = gt | (eq & (pe <= need))
    mask_ref[0] = sel.astype(jnp.uint8)

    pos0 = pg + jnp.minimum(pe, need) - 1
    dist = jnp.where(sel, cols - pos0, 0)
    d2 = (dist << 1) | sel.astype(jnp.int32)
    idx = cols
    s = 1
    while s < P:
        rd2 = _rollL(d2, s)
        take = ((rd2 & 1) == 1) & (((rd2 >> 1) & s) != 0)
        d2 = jnp.where(take, rd2 & ~(2 * s), d2)
        key = jnp.where(take, _rollL(key, s), key)
        idx = jnp.where(take, _rollL(idx, s), idx)
        moved = ((d2 & 1) == 1) & (((d2 >> 1) & s) != 0) & (~take)
        d2 = jnp.where(moved, 0, d2)
        s *= 2
    skey = key[:, :KSEL]
    sidx = idx[:, :KSEL]

    colk = jax.lax.broadcasted_iota(jnp.int32, (R, KSEL), 1)
    for p in range(1, 10):
        for j in range(p - 1, -1, -1):
            sft = 1 << j
            lower = (colk & sft) == 0
            pk = jnp.where(lower, _rollL(skey, sft), _rollR(skey, sft))
            pi = jnp.where(lower, _rollL(sidx, sft), _rollR(sidx, sft))
            g = (pk > skey) | ((pk == skey) & (pi < sidx))
            take = g ^ (~lower)
            if p < 9:
                take = take ^ (((colk >> p) & 1) == 1)
            skey = jnp.where(take, pk, skey)
            sidx = jnp.where(take, pi, sidx)
    idx_ref[0] = sidx


def _run_slab(index_scores, q0, rows, P):
    grid = (B, rows // R)
    kern = functools.partial(_topk_block_kernel, P=P, q0=q0)
    mask_u8, idx = pl.pallas_call(
        kern,
        grid=grid,
        in_specs=[
            pl.BlockSpec((1, R, P), lambda b, i, q0r=q0 // R: (b, q0r + i, 0)),
        ],
        out_specs=[
            pl.BlockSpec((1, R, P), lambda b, i: (b, i, 0)),
            pl.BlockSpec((1, R, KSEL), lambda b, i: (b, i, 0)),
        ],
        out_shape=[
            jax.ShapeDtypeStruct((B, rows, P), jnp.uint8),
            jax.ShapeDtypeStruct((B, rows, KSEL), jnp.int32),
        ],
    )(index_scores)
    if P < K:
        mask_u8 = jnp.concatenate(
            [mask_u8, jnp.zeros((B, rows, K - P), jnp.uint8)], axis=2)
    return mask_u8, idx


def kernel(index_scores):
    slabs = [(0, 512, 512), (512, 512, 1024), (1024, 1024, 2048),
             (2048, 2048, 4096)]
    mask_parts, idx_parts = [], []
    for q0, rows, P in slabs:
        m_u8, idx = _run_slab(index_scores, q0, rows, P)
        mask_parts.append(m_u8)
        idx_parts.append(idx)
    top_k_mask = jnp.concatenate(mask_parts, axis=1) != 0
    top_k_indices = jnp.concatenate(idx_parts, axis=1)
    stats = jnp.asarray(
        [1.0 - KSEL / K, float(KSEL), KSEL / K], dtype=jnp.float32)
    return (top_k_mask, top_k_indices, stats)
